# baseline (speedup 1.0000x reference)
import functools

import jax
import jax.numpy as jnp
from jax import lax
from jax.experimental import pallas as pl
from jax.experimental.pallas import tpu as pltpu
from jax.experimental.pallas import tpu_sc as plsc

BATCH = 1024
MAX_LEN = 200
D = 128
NSL = D // 16

_info = plsc.get_sparse_core_info()
_NC, _NS = _info.num_cores, _info.num_subcores
NW = _NC * _NS
ROWS = BATCH * MAX_LEN
RPW = ROWS // NW
SPW = RPW // MAX_LEN
NBUF = 2


def _pe_body(o_ref):
    pos = lax.broadcasted_iota(jnp.int32, (MAX_LEN, D), 0).astype(jnp.float32)
    di = lax.broadcasted_iota(jnp.int32, (MAX_LEN, D), 1)
    deven = ((di // 2) * 2).astype(jnp.float32)
    ang = pos * jnp.exp(-(jnp.log(10000.0) / D) * deven)
    o_ref[...] = jnp.where(di % 2 == 0, jnp.sin(ang), jnp.cos(ang))


_pe_call = pl.pallas_call(
    _pe_body, out_shape=jax.ShapeDtypeStruct((MAX_LEN, D), jnp.float32))


_mesh = plsc.VectorSubcoreMesh(core_axis_name="c", subcore_axis_name="s")


@functools.partial(
    pl.kernel,
    mesh=_mesh,
    out_type=jax.ShapeDtypeStruct((ROWS, D), jnp.float32),
    scratch_types=[
        pltpu.VMEM((SPW, MAX_LEN), jnp.int32),
        pltpu.VMEM((D, D), jnp.float32),
        pltpu.VMEM((MAX_LEN, D), jnp.float32),
        pltpu.VMEM((NBUF, MAX_LEN, D), jnp.float32),
    ] + [pltpu.SemaphoreType.DMA] * NBUF,
)
def _sc_embed(idx_hbm, table_hbm, pe_hbm, out_hbm,
              idx_v, table_v, pe_v, bufs, *ssem):
    wid = lax.axis_index("s") * _NC + lax.axis_index("c")
    pltpu.sync_copy(idx_hbm.at[pl.ds(wid * SPW, SPW)], idx_v)
    pltpu.sync_copy(table_hbm, table_v)
    pltpu.sync_copy(pe_hbm, pe_v)
    wbase = wid * RPW

    def start_scatter(s, b):
        pltpu.async_copy(
            bufs.at[b], out_hbm.at[pl.ds(wbase + s * MAX_LEN, MAX_LEN)],
            ssem[b])

    def wait_scatter(b):
        pltpu.make_async_copy(
            bufs.at[b], out_hbm.at[pl.ds(0, MAX_LEN)], ssem[b]).wait()

    def build_sentence(b, s):
        def emit_rows(r0, idvec, lanes):
            for u in lanes:
                rr = r0 + u
                rid = idvec[u]
                for c in range(NSL):
                    sl = pl.ds(c * 16, 16)
                    bufs[b, rr, sl] = table_v[rid, sl] + pe_v[rr, sl]

        def rows16(k, rc):
            r0 = k * 16
            emit_rows(r0, idx_v[s, pl.ds(r0, 16)], range(16))
            return rc

        lax.fori_loop(0, MAX_LEN // 16, rows16, 0)
        emit_rows(MAX_LEN - 16, idx_v[s, pl.ds(MAX_LEN - 16, 16)],
                  range(8, 16))

    def body(s, carry):
        b = s % NBUF

        @pl.when(s >= NBUF)
        def _():
            @pl.when(b == 0)
            def _():
                wait_scatter(0)

            @pl.when(b == 1)
            def _():
                wait_scatter(1)

        build_sentence(b, s)

        @pl.when(b == 0)
        def _():
            start_scatter(s, 0)

        @pl.when(b == 1)
        def _():
            start_scatter(s, 1)

        return carry

    lax.fori_loop(0, SPW, body, 0)
    wait_scatter(0)
    wait_scatter(1)


def kernel(batch, table):
    pe = _pe_call()
    idx = batch.astype(jnp.int32)
    out = _sc_embed(idx, table, pe)
    return out.reshape(BATCH, MAX_LEN, D)


# speedup vs baseline: 2.4121x; 2.4121x over previous
import functools

import jax
import jax.numpy as jnp
from jax import lax
from jax.experimental import pallas as pl
from jax.experimental.pallas import tpu as pltpu
from jax.experimental.pallas import tpu_sc as plsc

BATCH = 1024
MAX_LEN = 200
D = 128
NSL = D // 16

_info = plsc.get_sparse_core_info()
_NC, _NS = _info.num_cores, _info.num_subcores
NW = _NC * _NS
ROWS = BATCH * MAX_LEN
RPW = ROWS // NW
SPW = RPW // MAX_LEN
CH = MAX_LEN // 2
NBUF = 3


def _pe_body(o_ref):
    pos = lax.broadcasted_iota(jnp.int32, (MAX_LEN, D), 0).astype(jnp.float32)
    di = lax.broadcasted_iota(jnp.int32, (MAX_LEN, D), 1)
    deven = ((di // 2) * 2).astype(jnp.float32)
    ang = pos * jnp.exp(-(jnp.log(10000.0) / D) * deven)
    o_ref[...] = jnp.where(di % 2 == 0, jnp.sin(ang), jnp.cos(ang))


_pe_call = pl.pallas_call(
    _pe_body, out_shape=jax.ShapeDtypeStruct((MAX_LEN, D), jnp.float32))


def _rep_body(t_ref, o_ref):
    o_ref[...] = t_ref[...]


_rep_call = pl.pallas_call(
    _rep_body,
    grid=(NW,),
    in_specs=[pl.BlockSpec((D, D), lambda i: (0, 0))],
    out_specs=pl.BlockSpec((D, D), lambda i: (i, 0)),
    out_shape=jax.ShapeDtypeStruct((NW * D, D), jnp.float32))


_mesh = plsc.VectorSubcoreMesh(core_axis_name="c", subcore_axis_name="s")


@functools.partial(
    pl.kernel,
    mesh=_mesh,
    out_type=jax.ShapeDtypeStruct((ROWS, D), jnp.float32),
    scratch_types=[
        pltpu.VMEM((SPW, MAX_LEN), jnp.int32),
        pltpu.VMEM((MAX_LEN, D), jnp.float32),
        pltpu.VMEM((NBUF, MAX_LEN, D), jnp.float32),
        pltpu.VMEM((NBUF, CH), jnp.int32),
        pltpu.VMEM((NBUF, CH), jnp.int32),
    ] + [pltpu.SemaphoreType.DMA] * (2 * NBUF),
)
def _sc_embed(idx_hbm, trep_hbm, pe_hbm, out_hbm,
              idx_v, pe_v, bufs, adj_a, adj_b, *sems):
    gsem = sems[:NBUF]
    ssem = sems[NBUF:]
    wid = lax.axis_index("s") * _NC + lax.axis_index("c")
    pltpu.sync_copy(idx_hbm.at[pl.ds(wid * SPW, SPW)], idx_v)
    pltpu.sync_copy(pe_hbm, pe_v)
    wbase = wid * RPW
    rebase = wid * D

    def prep_ids(s, b):
        for half, adj in ((0, adj_a), (1, adj_b)):
            for k in range(0, CH, 16):
                o = min(k, CH - 16)
                sl = pl.ds(half * CH + o, 16)
                adj[b, pl.ds(o, 16)] = idx_v[s, sl] + rebase

    def start_gathers(b):
        pltpu.async_copy(
            trep_hbm.at[adj_a.at[b]], bufs.at[b, pl.ds(0, CH)], gsem[b])
        pltpu.async_copy(
            trep_hbm.at[adj_b.at[b]], bufs.at[b, pl.ds(CH, CH)], gsem[b])

    def wait_gathers(b):
        pltpu.make_async_copy(
            trep_hbm.at[adj_a.at[b]], bufs.at[b, pl.ds(0, CH)],
            gsem[b]).wait()
        pltpu.make_async_copy(
            trep_hbm.at[adj_b.at[b]], bufs.at[b, pl.ds(CH, CH)],
            gsem[b]).wait()

    def start_scatter(s, b):
        pltpu.async_copy(
            bufs.at[b], out_hbm.at[pl.ds(wbase + s * MAX_LEN, MAX_LEN)],
            ssem[b])

    def wait_scatter(b):
        pltpu.make_async_copy(
            bufs.at[b], out_hbm.at[pl.ds(0, MAX_LEN)], ssem[b]).wait()

    def add_pe(b):
        def row(r, rc):
            for u in range(2):
                for c in range(NSL):
                    sl = pl.ds(c * 16, 16)
                    plsc.addupdate(
                        bufs.at[b, 2 * r + u, sl], pe_v[2 * r + u, sl])
            return rc

        lax.fori_loop(0, MAX_LEN // 2, row, 0)

    def slot(s, b, wait_prev, next_s):
        bn = (b + 1) % NBUF
        if wait_prev:
            wait_scatter(bn)
        if next_s is not None:
            prep_ids(next_s, bn)
            start_gathers(bn)
        wait_gathers(b)
        add_pe(b)
        start_scatter(s, b)

    prep_ids(0, 0)
    start_gathers(0)
    slot(0, 0, False, 1)
    slot(1, 1, False, 2)
    slot(2, 2, True, 3)

    def body(g, carry):
        s0 = 3 * g + 3
        for b in range(NBUF):
            slot(s0 + b, b, True, s0 + b + 1)
        return carry

    lax.fori_loop(0, (SPW - 5) // 3, body, 0)

    slot(SPW - 2, (SPW - 2) % NBUF, True, SPW - 1)
    slot(SPW - 1, (SPW - 1) % NBUF, True, None)
    wait_scatter((SPW - 2) % NBUF)
    wait_scatter((SPW - 1) % NBUF)


def kernel(batch, table):
    pe = _pe_call()
    trep = _rep_call(table)
    idx = batch.astype(jnp.int32)
    out = _sc_embed(idx, trep, pe)
    return out.reshape(BATCH, MAX_LEN, D)


# speedup vs baseline: 3.1161x; 1.2919x over previous
import functools

import jax
import jax.numpy as jnp
from jax import lax
from jax.experimental import pallas as pl
from jax.experimental.pallas import tpu as pltpu
from jax.experimental.pallas import tpu_sc as plsc

BATCH = 1024
MAX_LEN = 200
D = 128
NSL = D // 16

_info = plsc.get_sparse_core_info()
_NC, _NS = _info.num_cores, _info.num_subcores
NW = _NC * _NS
ROWS = BATCH * MAX_LEN
RPW = ROWS // NW
SPW = RPW // MAX_LEN
CH = MAX_LEN // 2
NBUF = 3


def _pe_body(o_ref):
    pos = lax.broadcasted_iota(jnp.int32, (MAX_LEN, D), 0).astype(jnp.float32)
    di = lax.broadcasted_iota(jnp.int32, (MAX_LEN, D), 1)
    deven = ((di // 2) * 2).astype(jnp.float32)
    ang = pos * jnp.exp(-(jnp.log(10000.0) / D) * deven)
    o_ref[...] = jnp.where(di % 2 == 0, jnp.sin(ang), jnp.cos(ang))


_pe_call = pl.pallas_call(
    _pe_body, out_shape=jax.ShapeDtypeStruct((MAX_LEN, D), jnp.float32))


_mesh = plsc.VectorSubcoreMesh(core_axis_name="c", subcore_axis_name="s")


@functools.partial(
    pl.kernel,
    mesh=_mesh,
    out_type=jax.ShapeDtypeStruct((ROWS, D), jnp.float32),
    scratch_types=[
        pltpu.VMEM((SPW, MAX_LEN), jnp.int32),
        pltpu.VMEM((MAX_LEN, D), jnp.float32),
        pltpu.VMEM((NBUF, MAX_LEN, D), jnp.float32),
        pltpu.VMEM((NBUF, CH), jnp.int32),
        pltpu.VMEM((NBUF, CH), jnp.int32),
        pltpu.VMEM_SHARED((_NS * D, D), jnp.float32),
    ] + [pltpu.SemaphoreType.DMA] * (2 * NBUF),
)
def _sc_embed(idx_hbm, table_hbm, pe_hbm, out_hbm,
              idx_v, pe_v, bufs, adj_a, adj_b, shared, *sems):
    gsem = sems[:NBUF]
    ssem = sems[NBUF:]
    sid = lax.axis_index("s")
    wid = sid * _NC + lax.axis_index("c")
    pltpu.sync_copy(idx_hbm.at[pl.ds(wid * SPW, SPW)], idx_v)
    pltpu.sync_copy(pe_hbm, pe_v)
    pltpu.sync_copy(table_hbm, shared.at[pl.ds(sid * D, D)])
    plsc.subcore_barrier()
    wbase = wid * RPW
    rebase = sid * D

    def prep_ids(s, b):
        for half, adj in ((0, adj_a), (1, adj_b)):
            for k in range(0, CH, 16):
                o = min(k, CH - 16)
                sl = pl.ds(half * CH + o, 16)
                adj[b, pl.ds(o, 16)] = idx_v[s, sl] + rebase

    def start_gathers(b):
        pltpu.async_copy(
            shared.at[adj_a.at[b]], bufs.at[b, pl.ds(0, CH)], gsem[b])
        pltpu.async_copy(
            shared.at[adj_b.at[b]], bufs.at[b, pl.ds(CH, CH)], gsem[b])

    def wait_gathers(b):
        pltpu.make_async_copy(
            shared.at[adj_a.at[b]], bufs.at[b, pl.ds(0, CH)],
            gsem[b]).wait()
        pltpu.make_async_copy(
            shared.at[adj_b.at[b]], bufs.at[b, pl.ds(CH, CH)],
            gsem[b]).wait()

    def start_scatter(s, b):
        pltpu.async_copy(
            bufs.at[b], out_hbm.at[pl.ds(wbase + s * MAX_LEN, MAX_LEN)],
            ssem[b])

    def wait_scatter(b):
        pltpu.make_async_copy(
            bufs.at[b], out_hbm.at[pl.ds(0, MAX_LEN)], ssem[b]).wait()

    def add_pe(b):
        def row(r, rc):
            for u in range(2):
                for c in range(NSL):
                    sl = pl.ds(c * 16, 16)
                    plsc.addupdate(
                        bufs.at[b, 2 * r + u, sl], pe_v[2 * r + u, sl])
            return rc

        lax.fori_loop(0, MAX_LEN // 2, row, 0)

    def slot(s, b, wait_prev, next_s):
        bn = (b + 1) % NBUF
        if wait_prev:
            wait_scatter(bn)
        if next_s is not None:
            prep_ids(next_s, bn)
            start_gathers(bn)
        wait_gathers(b)
        add_pe(b)
        start_scatter(s, b)

    prep_ids(0, 0)
    start_gathers(0)
    slot(0, 0, False, 1)
    slot(1, 1, False, 2)
    slot(2, 2, True, 3)

    def body(g, carry):
        s0 = 3 * g + 3
        for b in range(NBUF):
            slot(s0 + b, b, True, s0 + b + 1)
        return carry

    lax.fori_loop(0, (SPW - 5) // 3, body, 0)

    slot(SPW - 2, (SPW - 2) % NBUF, True, SPW - 1)
    slot(SPW - 1, (SPW - 1) % NBUF, True, None)
    wait_scatter((SPW - 2) % NBUF)
    wait_scatter((SPW - 1) % NBUF)


def kernel(batch, table):
    pe = _pe_call()
    idx = batch.astype(jnp.int32)
    out = _sc_embed(idx, table, pe)
    return out.reshape(BATCH, MAX_LEN, D)
